# Initial kernel scaffold; baseline (speedup 1.0000x reference)
import jax
import jax.numpy as jnp
from jax.experimental import pallas as pl


def kernel(atom_type, edge_index, edge_type, bond_anchor, bond_inbound, angle_deltas, atom_emb, bond_emb, l_atom_emb, r_atom_emb, orig_emb, dest_emb, l_bond_emb, r_bond_emb, anchor_emb, ring_W, ring_b, comb_W, comb_b, edge_W0, edge_b0, edge_W1, edge_b1, node_W0, node_b0, node_W1, node_b1):
    raise NotImplementedError("write your pallas kernel here")


# speedup vs baseline: 12.1662x; 12.1662x over previous
import functools

import jax
import jax.numpy as jnp
from jax import lax
from jax.experimental import pallas as pl
from jax.experimental.pallas import tpu as pltpu
from jax.experimental.pallas import tpu_sc as plsc

NC, NS, L = 2, 16, 16
_SC_PARAMS = pltpu.CompilerParams(needs_layout_passes=False,
                                  use_tc_tiling_on_sc=False)
NW = NC * NS
N = 10000
E = 320000
A = 320000
EP = 327680
AP = 327680
EH = EP // 2
SG_DUMMY = 2048
SG_ROWS = EH + SG_DUMMY
NH = N // 2
P_SC_PAD = 524288
F32 = jnp.float32
I32 = jnp.int32


def _mesh():
    return plsc.VectorSubcoreMesh(
        core_axis_name="c", subcore_axis_name="s", num_cores=NC, num_subcores=NS)


def _wid():
    return lax.axis_index("s") * NC + lax.axis_index("c")


def _iota16():
    return lax.iota(I32, 16)


def _zero2d(ref, nrows, ncols):
    total = nrows * ncols
    assert total % 16 == 0
    zval = jnp.zeros((16,), ref.dtype)
    lane = _iota16()

    def body(i, _):
        flat = i * 16 + lane
        plsc.store_scatter(ref, [flat // ncols, flat % ncols], zval)
        return 0

    lax.fori_loop(0, total // 16, body, 0)


def _k1_body(atom_emb, bond_emb, l_atom_emb, r_atom_emb, orig_emb, dest_emb,
             l_bond_emb, r_bond_emb, anchor_emb, ring_W, ring_b, comb_W,
             comb_b, eW0, eW1, eb0, eb1, nW0,
             tla_o, tra_o, tao_o, td_o, tb_o, w8_o, awa_o, awb_o):
    f = lambda r: r[...]
    dot = functools.partial(jnp.dot, preferred_element_type=F32)
    eW = jnp.concatenate([f(eW0), f(eW1)], axis=1)
    W2 = dot(f(comb_W)[32:64, :], eW)
    TL8 = dot(dot(f(l_bond_emb), f(ring_W)[0:32]), W2)
    TRb8 = dot(dot(f(r_bond_emb), f(ring_W)[32:64]), W2)
    TAn8 = dot(dot(f(anchor_emb), f(ring_W)[64:96]), W2)
    TLa8 = dot(dot(f(l_atom_emb), f(ring_W)[96:128]), W2)
    TRa8 = dot(dot(f(r_atom_emb), f(ring_W)[128:160]), W2)
    TO8 = dot(dot(f(orig_emb), f(ring_W)[160:192]), W2)
    TD8 = dot(dot(f(dest_emb), f(ring_W)[192:224]), W2)
    w8 = dot(f(ring_W)[224:225, :], W2)
    rb8 = dot(f(ring_b)[None, :], W2)
    r1600 = lax.broadcasted_iota(I32, (1600, 16), 0)
    rep16 = jnp.where(r1600 // 100 == lax.broadcasted_iota(I32, (1600, 16), 1),
                      1.0, 0.0).astype(F32)
    r1600b = lax.broadcasted_iota(I32, (1600, 100), 0)
    tile100 = jnp.where(r1600b % 100 == lax.broadcasted_iota(I32, (1600, 100), 1),
                        1.0, 0.0).astype(F32)
    tla_o[...] = dot(rep16, TL8) + dot(tile100, TLa8)
    tra_o[...] = dot(rep16, TRb8) + dot(tile100, TRa8)
    tao_o[...] = dot(rep16, TAn8) + dot(tile100, TO8) + rb8
    td_o[...] = TD8
    c8 = (dot(f(comb_b)[None, :], eW)
          + jnp.concatenate([f(eb0), f(eb1)])[None, :])
    tb_o[...] = dot(dot(f(bond_emb), f(comb_W)[0:32, :]), eW) + c8
    w8_o[...] = jnp.concatenate([w8, jnp.zeros((1, 8), F32)], axis=1)
    AW = [dot(f(atom_emb), f(nW0)[i]) for i in range(4)]
    ri = lax.broadcasted_iota(I32, (200, 100), 0)
    ci = lax.broadcasted_iota(I32, (200, 100), 1)
    sel0 = jnp.where(ri == 2 * ci, 1.0, 0.0).astype(F32)
    sel1 = jnp.where(ri == 2 * ci + 1, 1.0, 0.0).astype(F32)
    awa_o[...] = dot(sel0, AW[0]) + dot(sel1, AW[1])
    awb_o[...] = dot(sel0, AW[2]) + dot(sel1, AW[3])


def _k1(atom_emb, bond_emb, l_atom_emb, r_atom_emb, orig_emb, dest_emb,
        l_bond_emb, r_bond_emb, anchor_emb, ring_W, ring_b, comb_W, comb_b,
        eW0, eW1, eb0, eb1, nW0):
    outs = (
        jax.ShapeDtypeStruct((1600, 8), F32),
        jax.ShapeDtypeStruct((1600, 8), F32),
        jax.ShapeDtypeStruct((1600, 8), F32),
        jax.ShapeDtypeStruct((100, 8), F32),
        jax.ShapeDtypeStruct((16, 8), F32),
        jax.ShapeDtypeStruct((1, 16), F32),
        jax.ShapeDtypeStruct((200, 64), F32),
        jax.ShapeDtypeStruct((200, 64), F32),
    )
    return pl.pallas_call(_k1_body, out_shape=outs)(
        atom_emb, bond_emb, l_atom_emb, r_atom_emb, orig_emb, dest_emb,
        l_bond_emb, r_bond_emb, anchor_emb, ring_W, ring_b, comb_W, comb_b,
        eW0, eW1, eb0, eb1, nW0)


def _k2_body(atype_h, src_h, dst_h, et_h, tla_h, tra_h, tao_h, td_h,
             as_h, gl_h, gr_h, ga_h,
             atv, tlav, trav, taov, tdv,
             srcv, dstv, etv, asv, pkv, adv, glv, grv, gav):
    base = _wid() * (EP // NW)
    lane = _iota16()
    pltpu.sync_copy(atype_h, atv)
    pltpu.sync_copy(tla_h, tlav)
    pltpu.sync_copy(tra_h, trav)
    pltpu.sync_copy(tao_h, taov)
    pltpu.sync_copy(td_h, tdv)

    def chunk(kc, _):
        off = base + kc * 1024
        pltpu.sync_copy(src_h.at[pl.ds(off, 1024)], srcv)
        pltpu.sync_copy(dst_h.at[pl.ds(off, 1024)], dstv)
        pltpu.sync_copy(et_h.at[pl.ds(off, 1024)], etv)

        def body(i, _):
            sl = pl.ds(i * 16, 16)
            a_s = plsc.load_gather(atv, [srcv[sl]])
            a_d = plsc.load_gather(atv, [dstv[sl]])
            asv[sl] = a_s
            pkv[sl] = etv[sl] * 100 + a_s
            adv[sl] = a_d
            return 0

        lax.fori_loop(0, 64, body, 0)

        def gbody(f, _):
            r16 = 2 * f + lane // 8
            c16 = lane % 8
            fi = plsc.load_gather(pkv, [r16]) * 8 + c16
            di = plsc.load_gather(adv, [r16]) * 8 + c16
            plsc.store_scatter(glv, [r16, c16], plsc.load_gather(tlav, [fi]))
            plsc.store_scatter(grv, [r16, c16], plsc.load_gather(trav, [fi]))
            plsc.store_scatter(gav, [r16, c16],
                               plsc.load_gather(taov, [fi])
                               + plsc.load_gather(tdv, [di]))
            return 0

        lax.fori_loop(0, 512, gbody, 0)
        pltpu.sync_copy(asv, as_h.at[pl.ds(off, 1024)])
        pltpu.sync_copy(glv, gl_h.at[pl.ds(off, 1024)])
        pltpu.sync_copy(grv, gr_h.at[pl.ds(off, 1024)])
        pltpu.sync_copy(gav, ga_h.at[pl.ds(off, 1024)])
        return 0

    lax.fori_loop(0, 10, chunk, 0)


def _k2(atype, src_p, dst_p, et_p, tla_f, tra_f, tao_f, td_f):
    outs = (jax.ShapeDtypeStruct((EP,), I32),
            jax.ShapeDtypeStruct((EP, 8), F32),
            jax.ShapeDtypeStruct((EP, 8), F32),
            jax.ShapeDtypeStruct((EP, 8), F32))
    scratch = [
        pltpu.VMEM((N,), I32),
        pltpu.VMEM((12800,), F32), pltpu.VMEM((12800,), F32),
        pltpu.VMEM((12800,), F32), pltpu.VMEM((800,), F32),
        pltpu.VMEM((1024,), I32), pltpu.VMEM((1024,), I32),
        pltpu.VMEM((1024,), I32), pltpu.VMEM((1024,), I32),
        pltpu.VMEM((1024,), I32), pltpu.VMEM((1024,), I32),
        pltpu.VMEM((1024, 8), F32), pltpu.VMEM((1024, 8), F32),
        pltpu.VMEM((1024, 8), F32),
    ]
    return pl.kernel(_k2_body, out_type=outs, mesh=_mesh(),
                     scratch_types=scratch, compiler_params=_SC_PARAMS)(
        atype, src_p, dst_p, et_p, tla_f, tra_f, tao_f, td_f)


def _k3_body(anc_h, bi0_h, bi1_h, dlt_h, gl_h, gr_h, ga_h, et_h, tb_h, w8_h,
             attr_h,
             sg, tbv, w8v, ancv, bi0v, bi1v, dltv,
             glb, grb, gab, dwb, six, outb, etb):
    cid = lax.axis_index("c")
    sid = lax.axis_index("s")
    lane = _iota16()
    lo = cid * EH

    pltpu.sync_copy(tb_h, tbv)
    pltpu.sync_copy(w8_h, w8v)

    _zero2d(outb, 2048, 8)

    def zchunk(i, _):
        pltpu.sync_copy(outb, sg.at[pl.ds(sid * 10368 + i * 2048, 2048)])
        return 0

    lax.fori_loop(0, 5, zchunk, 0)
    pltpu.sync_copy(outb.at[pl.ds(0, 128)],
                    sg.at[pl.ds(sid * 10368 + 10240, 128)])
    plsc.subcore_barrier()

    w8c = plsc.load_gather(w8v, [lane % 8])
    abase = sid * (AP // NS)

    def achunk(kc, _):
        off = abase + kc * 2048
        pltpu.sync_copy(anc_h.at[pl.ds(off, 2048)], ancv)
        pltpu.sync_copy(bi0_h.at[pl.ds(off, 2048)], bi0v)
        pltpu.sync_copy(bi1_h.at[pl.ds(off, 2048)], bi1v)
        pltpu.sync_copy(dlt_h.at[pl.ds(off, 2048)], dltv)

        def group(g, _):
            goff = g * 128
            pltpu.sync_copy(gl_h.at[bi0v.at[pl.ds(goff, 128)]], glb)
            pltpu.sync_copy(gr_h.at[bi1v.at[pl.ds(goff, 128)]], grb)
            pltpu.sync_copy(ga_h.at[ancv.at[pl.ds(goff, 128)]], gab)

            def sub(s, _):
                sl = pl.ds(goff + s * 16, 16)
                a16 = ancv[sl]
                inr = (a16 >= lo) & (a16 < lo + EH)
                dummy = EH + ((goff + s * 16 + lane) & (SG_DUMMY - 1))
                plsc.store_scatter(six, [s * 16 + lane],
                                   jnp.where(inr, a16 - lo, dummy))
                return 0

            lax.fori_loop(0, 8, sub, 0)

            def dsub(f, _):
                r16 = 2 * f + lane // 8
                d16 = plsc.load_gather(dltv, [goff + r16])
                plsc.store_scatter(dwb, [r16, lane % 8], d16 * w8c)
                return 0

            lax.fori_loop(0, 64, dsub, 0)
            pltpu.sync_copy(glb, sg.at[six], add=True)
            pltpu.sync_copy(grb, sg.at[six], add=True)
            pltpu.sync_copy(gab, sg.at[six], add=True)
            pltpu.sync_copy(dwb, sg.at[six], add=True)
            return 0

        lax.fori_loop(0, 16, group, 0)
        return 0

    lax.fori_loop(0, 10, achunk, 0)
    plsc.subcore_barrier()

    loff0 = sid * (EH // NS)
    goff0 = lo + loff0

    def bchunk(kc, _):
        loff = loff0 + kc * 2048
        goff = goff0 + kc * 2048
        pltpu.sync_copy(sg.at[pl.ds(loff, 2048)], outb)
        pltpu.sync_copy(et_h.at[pl.ds(goff, 2048)], etb)

        def sub(f, _):
            r16 = 2 * f + lane // 8
            c16 = lane % 8
            e16 = plsc.load_gather(etb, [r16])
            val = (plsc.load_gather(outb, [r16, c16])
                   + plsc.load_gather(tbv, [e16 * 8 + c16]))
            val = jnp.where(goff + r16 < E, val, 0.0)
            plsc.store_scatter(outb, [r16, c16], val)
            return 0

        lax.fori_loop(0, 1024, sub, 0)
        pltpu.sync_copy(outb, attr_h.at[pl.ds(goff, 2048)])
        return 0

    lax.fori_loop(0, 5, bchunk, 0)


def _k3(anc_p, bi0_p, bi1_p, dlt_p, gl, gr, ga, et_p, tb_f, w8_f):
    scratch = [
        pltpu.VMEM_SHARED((SG_ROWS, 8), F32),
        pltpu.VMEM((128,), F32), pltpu.VMEM((16,), F32),
        pltpu.VMEM((2048,), I32), pltpu.VMEM((2048,), I32),
        pltpu.VMEM((2048,), I32), pltpu.VMEM((2048,), F32),
        pltpu.VMEM((128, 8), F32), pltpu.VMEM((128, 8), F32),
        pltpu.VMEM((128, 8), F32), pltpu.VMEM((128, 8), F32),
        pltpu.VMEM((128,), I32),
        pltpu.VMEM((2048, 8), F32), pltpu.VMEM((2048,), I32),
    ]
    return pl.kernel(_k3_body,
                     out_type=jax.ShapeDtypeStruct((EP, 8), F32),
                     mesh=_mesh(), scratch_types=scratch,
                     compiler_params=_SC_PARAMS)(
        anc_p, bi0_p, bi1_p, dlt_p, gl, gr, ga, et_p, tb_f, w8_f)


def _k4_body(dst_h, as_h, attr_h, p_out,
             psh, dstv, asv, attrv, mb, pidx, pb, *, cp):
    cid = lax.axis_index("c")
    sid = lax.axis_index("s")
    lane = _iota16()
    nlo = cid * NH

    _zero2d(pb, 2048, 8)

    def zchunk(i, _):
        pltpu.sync_copy(pb, psh.at[pl.ds(sid * 8192 + i * 2048, 2048)])
        return 0

    lax.fori_loop(0, 4, zchunk, 0)
    plsc.subcore_barrier()

    ebase = sid * (EP // NS)

    def echunk(kc, _):
        off = ebase + kc * 2048
        pltpu.sync_copy(dst_h.at[pl.ds(off, 2048)], dstv)
        pltpu.sync_copy(as_h.at[pl.ds(off, 2048)], asv)
        pltpu.sync_copy(attr_h.at[pl.ds(off, 2048)], attrv)

        def group(g, _):
            goff = g * 128

            def sub(s, _):
                sl = pl.ds(goff + s * 16, 16)
                d16 = dstv[sl]
                as16 = asv[sl]
                row16 = s * 16 + lane
                inr = (d16 >= nlo) & (d16 < nlo + NH)
                r = (d16 - nlo) * 100 + as16
                q = jnp.where(inr, r // 4,
                              125000 + ((goff + s * 16 + lane) & 4095))
                cb = (r % 4) * 2
                plsc.store_scatter(pidx, [row16], q)
                lrow = goff + s * 16 + lane
                av0 = plsc.load_gather(
                    attrv, [lrow, jnp.full((16,), 2 * cp, I32)])
                av1 = plsc.load_gather(
                    attrv, [lrow, jnp.full((16,), 2 * cp + 1, I32)])
                for c in range(8):
                    cc = jnp.full((16,), c, I32)
                    val = (jnp.where(cb == c, av0, 0.0)
                           + jnp.where(cb + 1 == c, av1, 0.0))
                    plsc.store_scatter(mb, [row16, cc], val)
                return 0

            lax.fori_loop(0, 8, sub, 0)
            pltpu.sync_copy(mb, psh.at[pidx], add=True)
            return 0

        lax.fori_loop(0, 16, group, 0)
        return 0

    lax.fori_loop(0, 10, echunk, 0)
    plsc.subcore_barrier()

    dbase = sid * 8192

    def dchunk(i, _):
        off = dbase + i * 2048
        pltpu.sync_copy(psh.at[pl.ds(off, 2048)], pb)
        pltpu.sync_copy(pb, p_out.at[pl.ds(cid * 131072 + off, 2048)])
        return 0

    lax.fori_loop(0, 4, dchunk, 0)


def _k4(dst_p, asrc, attr8, cp):
    scratch = [
        pltpu.VMEM_SHARED((131072, 8), F32),
        pltpu.VMEM((2048,), I32), pltpu.VMEM((2048,), I32),
        pltpu.VMEM((2048, 8), F32),
        pltpu.VMEM((128, 8), F32), pltpu.VMEM((128,), I32),
        pltpu.VMEM((2048, 8), F32),
    ]
    body = functools.partial(_k4_body, cp=cp)
    return pl.kernel(body,
                     out_type=jax.ShapeDtypeStruct((2 * 131072, 8), F32),
                     mesh=_mesh(), scratch_types=scratch,
                     compiler_params=_SC_PARAMS)(dst_p, asrc, attr8)


def _k5_body(pa, pb, awa, awb, nb0, nW1, u_o):
    out1 = (jnp.dot(pa[...], awa[...], preferred_element_type=F32)
            + jnp.dot(pb[...], awb[...], preferred_element_type=F32)
            + jnp.sum(nb0[...], axis=0, keepdims=True))
    h1 = jnp.tanh(out1)
    vsum = jnp.sum(nW1[...], axis=-1)
    u_o[...] = lax.dot_general(h1, vsum, (((1,), (1,)), ((), ())),
                               preferred_element_type=F32)


def _k5(pa, pb, awa, awb, nb0, nW1):
    grid = (10,)
    return pl.pallas_call(
        _k5_body,
        grid=grid,
        in_specs=[
            pl.BlockSpec((1000, 200), lambda i: (i, 0)),
            pl.BlockSpec((1000, 200), lambda i: (i, 0)),
            pl.BlockSpec((200, 64), lambda i: (0, 0)),
            pl.BlockSpec((200, 64), lambda i: (0, 0)),
            pl.BlockSpec((4, 64), lambda i: (0, 0)),
            pl.BlockSpec((4, 64, 64), lambda i: (0, 0, 0)),
        ],
        out_specs=pl.BlockSpec((1000, 4), lambda i: (i, 0)),
        out_shape=jax.ShapeDtypeStruct((N, 4), F32),
    )(pa, pb, awa, awb, nb0, nW1)


def _k6_body(src_h, attr_h, u_h, part_h, uv, srcv, attrv, accv):
    wid = _wid()
    lane = _iota16()
    pltpu.sync_copy(u_h, uv)
    ebase = wid * (EP // NW)

    def chunk(kc, acc):
        off = ebase + kc * 2048
        pltpu.sync_copy(src_h.at[pl.ds(off, 2048)], srcv)
        pltpu.sync_copy(attr_h.at[pl.ds(off, 2048)], attrv)

        def body(i, acc):
            s16 = srcv[pl.ds(i * 16, 16)]
            row16 = i * 16 + lane
            for ii in range(4):
                ui = plsc.load_gather(uv, [s16, jnp.full((16,), ii, I32)])
                ai = plsc.load_gather(attrv,
                                      [row16, jnp.full((16,), 4 + ii, I32)])
                acc = acc + ui * ai
            return acc

        return lax.fori_loop(0, 128, body, acc)

    acc = lax.fori_loop(0, 5, chunk, jnp.zeros((16,), F32))
    accv[...] = acc
    pltpu.sync_copy(accv, part_h.at[wid])


def _k6(src_p, attr8, u):
    scratch = [
        pltpu.VMEM((N, 4), F32),
        pltpu.VMEM((2048,), I32), pltpu.VMEM((2048, 8), F32),
        pltpu.VMEM((16,), F32),
    ]
    return pl.kernel(_k6_body,
                     out_type=jax.ShapeDtypeStruct((NW, 16), F32),
                     mesh=_mesh(), scratch_types=scratch,
                     compiler_params=_SC_PARAMS)(src_p, attr8, u)


def _k7_body(part, nb1, out):
    out[...] = (jnp.sum(part[...]) / (64.0 * N)
                + jnp.sum(nb1[...]) / 64.0)[None, None]


def _k7(part, nb1):
    return pl.pallas_call(_k7_body,
                          out_shape=jax.ShapeDtypeStruct((1, 1), F32))(part, nb1)


def kernel(atom_type, edge_index, edge_type, bond_anchor, bond_inbound,
           angle_deltas, atom_emb, bond_emb, l_atom_emb, r_atom_emb, orig_emb,
           dest_emb, l_bond_emb, r_bond_emb, anchor_emb, ring_W, ring_b,
           comb_W, comb_b, edge_W0, edge_b0, edge_W1, edge_b1, node_W0,
           node_b0, node_W1, node_b1):
    atom_type = atom_type.astype(I32)
    src = edge_index[0].astype(I32)
    dst = edge_index[1].astype(I32)
    et = edge_type.astype(I32)
    anc = bond_anchor.astype(I32)
    bi0 = bond_inbound[:, 0].astype(I32)
    bi1 = bond_inbound[:, 1].astype(I32)

    epad = EP - E
    src_p = jnp.pad(src, (0, epad))
    dst_p = jnp.pad(dst, (0, epad))
    et_p = jnp.pad(et, (0, epad))
    apad = AP - A
    anc_p = jnp.pad(anc, (0, apad), constant_values=EP)
    bi0_p = jnp.pad(bi0, (0, apad))
    bi1_p = jnp.pad(bi1, (0, apad))
    dlt_p = jnp.pad(angle_deltas, (0, apad))

    tla, tra, tao, td, tb, w8, awa, awb = _k1(
        atom_emb, bond_emb, l_atom_emb, r_atom_emb, orig_emb, dest_emb,
        l_bond_emb, r_bond_emb, anchor_emb, ring_W, ring_b, comb_W, comb_b,
        edge_W0, edge_W1, edge_b0, edge_b1, node_W0)
    asrc, gl, gr, ga = _k2(atom_type, src_p, dst_p, et_p,
                           tla.reshape(-1), tra.reshape(-1), tao.reshape(-1),
                           td.reshape(-1))
    attr8 = _k3(anc_p, bi0_p, bi1_p, dlt_p, gl, gr, ga, et_p,
                tb.reshape(-1), w8.reshape(-1))
    pa4 = _k4(dst_p, asrc, attr8, 0)
    pb4 = _k4(dst_p, asrc, attr8, 1)
    def _unpad(p4):
        return jnp.concatenate(
            [p4[:125000], p4[131072:131072 + 125000]]).reshape(N, 200)
    pa = _unpad(pa4)
    pb = _unpad(pb4)
    u = _k5(pa, pb, awa, awb, node_b0, node_W1)
    part = _k6(src_p, attr8, u)
    out = _k7(part, node_b1)
    return out[0, 0]


# speedup vs baseline: 14.9123x; 1.2257x over previous
import functools

import jax
import jax.numpy as jnp
from jax import lax
from jax.experimental import pallas as pl
from jax.experimental.pallas import tpu as pltpu
from jax.experimental.pallas import tpu_sc as plsc

NC, NS, L = 2, 16, 16
_SC_PARAMS = pltpu.CompilerParams(needs_layout_passes=False,
                                  use_tc_tiling_on_sc=False)
NW = NC * NS
N = 10000
E = 320000
A = 320000
EP = 327680
AP = 327680
EH = EP // 2
SG_DUMMY = 2048
SG_ROWS = EH + SG_DUMMY
NH = N // 2
P_SC_PAD = 524288
F32 = jnp.float32
I32 = jnp.int32


def _mesh():
    return plsc.VectorSubcoreMesh(
        core_axis_name="c", subcore_axis_name="s", num_cores=NC, num_subcores=NS)


def _wid():
    return lax.axis_index("s") * NC + lax.axis_index("c")


def _iota16():
    return lax.iota(I32, 16)


def _zero2d(ref, nrows, ncols):
    total = nrows * ncols
    assert total % 16 == 0
    zval = jnp.zeros((16,), ref.dtype)
    lane = _iota16()

    def body(i, _):
        flat = i * 16 + lane
        plsc.store_scatter(ref, [flat // ncols, flat % ncols], zval)
        return 0

    lax.fori_loop(0, total // 16, body, 0)


def _k1_body(atom_emb, bond_emb, l_atom_emb, r_atom_emb, orig_emb, dest_emb,
             l_bond_emb, r_bond_emb, anchor_emb, ring_W, ring_b, comb_W,
             comb_b, eW0, eW1, eb0, eb1, nW0,
             tla_o, tra_o, tao_o, td_o, tb_o, w8_o, awa_o, awb_o):
    f = lambda r: r[...]
    dot = functools.partial(jnp.dot, preferred_element_type=F32)
    eW = jnp.concatenate([f(eW0), f(eW1)], axis=1)
    W2 = dot(f(comb_W)[32:64, :], eW)
    TL8 = dot(dot(f(l_bond_emb), f(ring_W)[0:32]), W2)
    TRb8 = dot(dot(f(r_bond_emb), f(ring_W)[32:64]), W2)
    TAn8 = dot(dot(f(anchor_emb), f(ring_W)[64:96]), W2)
    TLa8 = dot(dot(f(l_atom_emb), f(ring_W)[96:128]), W2)
    TRa8 = dot(dot(f(r_atom_emb), f(ring_W)[128:160]), W2)
    TO8 = dot(dot(f(orig_emb), f(ring_W)[160:192]), W2)
    TD8 = dot(dot(f(dest_emb), f(ring_W)[192:224]), W2)
    w8 = dot(f(ring_W)[224:225, :], W2)
    rb8 = dot(f(ring_b)[None, :], W2)
    r1600 = lax.broadcasted_iota(I32, (1600, 16), 0)
    rep16 = jnp.where(r1600 // 100 == lax.broadcasted_iota(I32, (1600, 16), 1),
                      1.0, 0.0).astype(F32)
    r1600b = lax.broadcasted_iota(I32, (1600, 100), 0)
    tile100 = jnp.where(r1600b % 100 == lax.broadcasted_iota(I32, (1600, 100), 1),
                        1.0, 0.0).astype(F32)
    tla_o[...] = dot(rep16, TL8) + dot(tile100, TLa8)
    tra_o[...] = dot(rep16, TRb8) + dot(tile100, TRa8)
    tao_o[...] = dot(rep16, TAn8) + dot(tile100, TO8) + rb8
    td_o[...] = TD8
    c8 = (dot(f(comb_b)[None, :], eW)
          + jnp.concatenate([f(eb0), f(eb1)])[None, :])
    tb_o[...] = dot(dot(f(bond_emb), f(comb_W)[0:32, :]), eW) + c8
    w8_o[...] = jnp.concatenate([w8, jnp.zeros((1, 8), F32)], axis=1)
    AW = [dot(f(atom_emb), f(nW0)[i]) for i in range(4)]
    ri = lax.broadcasted_iota(I32, (200, 100), 0)
    ci = lax.broadcasted_iota(I32, (200, 100), 1)
    sel0 = jnp.where(ri == 2 * ci, 1.0, 0.0).astype(F32)
    sel1 = jnp.where(ri == 2 * ci + 1, 1.0, 0.0).astype(F32)
    awa_o[...] = dot(sel0, AW[0]) + dot(sel1, AW[1])
    awb_o[...] = dot(sel0, AW[2]) + dot(sel1, AW[3])


def _k1(atom_emb, bond_emb, l_atom_emb, r_atom_emb, orig_emb, dest_emb,
        l_bond_emb, r_bond_emb, anchor_emb, ring_W, ring_b, comb_W, comb_b,
        eW0, eW1, eb0, eb1, nW0):
    outs = (
        jax.ShapeDtypeStruct((1600, 8), F32),
        jax.ShapeDtypeStruct((1600, 8), F32),
        jax.ShapeDtypeStruct((1600, 8), F32),
        jax.ShapeDtypeStruct((100, 8), F32),
        jax.ShapeDtypeStruct((16, 8), F32),
        jax.ShapeDtypeStruct((1, 16), F32),
        jax.ShapeDtypeStruct((200, 64), F32),
        jax.ShapeDtypeStruct((200, 64), F32),
    )
    return pl.pallas_call(_k1_body, out_shape=outs)(
        atom_emb, bond_emb, l_atom_emb, r_atom_emb, orig_emb, dest_emb,
        l_bond_emb, r_bond_emb, anchor_emb, ring_W, ring_b, comb_W, comb_b,
        eW0, eW1, eb0, eb1, nW0)


def _k2_body(atype_h, src_h, dst_h, et_h, tla_h, tra_h, tao_h, td_h,
             as_h, gl_h, gr_h, ga_h,
             atv, tlav, trav, taov, tdv,
             srcv, dstv, etv, asv, pkv, adv, glv, grv, gav):
    base = _wid() * (EP // NW)
    lane = _iota16()
    pltpu.sync_copy(atype_h, atv)
    pltpu.sync_copy(tla_h, tlav)
    pltpu.sync_copy(tra_h, trav)
    pltpu.sync_copy(tao_h, taov)
    pltpu.sync_copy(td_h, tdv)

    def chunk(kc, _):
        off = base + kc * 1024
        pltpu.sync_copy(src_h.at[pl.ds(off, 1024)], srcv)
        pltpu.sync_copy(dst_h.at[pl.ds(off, 1024)], dstv)
        pltpu.sync_copy(et_h.at[pl.ds(off, 1024)], etv)

        def body(i, _):
            sl = pl.ds(i * 16, 16)
            a_s = plsc.load_gather(atv, [srcv[sl]])
            a_d = plsc.load_gather(atv, [dstv[sl]])
            asv[sl] = a_s
            pkv[sl] = etv[sl] * 100 + a_s
            adv[sl] = a_d
            return 0

        lax.fori_loop(0, 64, body, 0)

        def gbody(f, _):
            r16 = 2 * f + lane // 8
            c16 = lane % 8
            fi = plsc.load_gather(pkv, [r16]) * 8 + c16
            di = plsc.load_gather(adv, [r16]) * 8 + c16
            plsc.store_scatter(glv, [r16, c16], plsc.load_gather(tlav, [fi]))
            plsc.store_scatter(grv, [r16, c16], plsc.load_gather(trav, [fi]))
            plsc.store_scatter(gav, [r16, c16],
                               plsc.load_gather(taov, [fi])
                               + plsc.load_gather(tdv, [di]))
            return 0

        lax.fori_loop(0, 512, gbody, 0)
        pltpu.sync_copy(asv, as_h.at[pl.ds(off, 1024)])
        pltpu.sync_copy(glv, gl_h.at[pl.ds(off, 1024)])
        pltpu.sync_copy(grv, gr_h.at[pl.ds(off, 1024)])
        pltpu.sync_copy(gav, ga_h.at[pl.ds(off, 1024)])
        return 0

    lax.fori_loop(0, 10, chunk, 0)


def _k2(atype, src_p, dst_p, et_p, tla_f, tra_f, tao_f, td_f):
    outs = (jax.ShapeDtypeStruct((EP,), I32),
            jax.ShapeDtypeStruct((EP, 8), F32),
            jax.ShapeDtypeStruct((EP, 8), F32),
            jax.ShapeDtypeStruct((EP, 8), F32))
    scratch = [
        pltpu.VMEM((N,), I32),
        pltpu.VMEM((12800,), F32), pltpu.VMEM((12800,), F32),
        pltpu.VMEM((12800,), F32), pltpu.VMEM((800,), F32),
        pltpu.VMEM((1024,), I32), pltpu.VMEM((1024,), I32),
        pltpu.VMEM((1024,), I32), pltpu.VMEM((1024,), I32),
        pltpu.VMEM((1024,), I32), pltpu.VMEM((1024,), I32),
        pltpu.VMEM((1024, 8), F32), pltpu.VMEM((1024, 8), F32),
        pltpu.VMEM((1024, 8), F32),
    ]
    return pl.kernel(_k2_body, out_type=outs, mesh=_mesh(),
                     scratch_types=scratch, compiler_params=_SC_PARAMS)(
        atype, src_p, dst_p, et_p, tla_f, tra_f, tao_f, td_f)


def _k3_body(anc_h, bi0_h, bi1_h, dlt_h, gl_h, gr_h, ga_h, et_h, tb_h, w8_h,
             attr_h,
             sg, tbv, w8v, ancv, bi0v, bi1v, dltv,
             glb, grb, gab, dwb, six, outb, etb, semg, sems):
    cid = lax.axis_index("c")
    sid = lax.axis_index("s")
    lane = _iota16()
    lo = cid * EH

    pltpu.sync_copy(tb_h, tbv)
    pltpu.sync_copy(w8_h, w8v)

    _zero2d(outb, 2048, 8)

    def zchunk(i, _):
        pltpu.sync_copy(outb, sg.at[pl.ds(sid * 10368 + i * 2048, 2048)])
        return 0

    lax.fori_loop(0, 5, zchunk, 0)
    pltpu.sync_copy(outb.at[pl.ds(0, 128)],
                    sg.at[pl.ds(sid * 10368 + 10240, 128)])
    plsc.subcore_barrier()

    w8c = plsc.load_gather(w8v, [lane % 8])
    abase = sid * (AP // NS)

    def _drain_scatters():
        pltpu.make_async_copy(glb, sg.at[six], sems).wait()
        pltpu.make_async_copy(grb, sg.at[six], sems).wait()
        pltpu.make_async_copy(gab, sg.at[six], sems).wait()
        pltpu.make_async_copy(dwb, sg.at[six], sems).wait()

    def achunk(kc, _):
        off = abase + kc * 2048
        pltpu.sync_copy(anc_h.at[pl.ds(off, 2048)], ancv)
        pltpu.sync_copy(bi0_h.at[pl.ds(off, 2048)], bi0v)
        pltpu.sync_copy(bi1_h.at[pl.ds(off, 2048)], bi1v)
        pltpu.sync_copy(dlt_h.at[pl.ds(off, 2048)], dltv)

        def group(g, _):
            goff = g * 128
            @pl.when(g > 0)
            def _():
                _drain_scatters()

            d1 = pltpu.async_copy(gl_h.at[bi0v.at[pl.ds(goff, 128)]], glb,
                                  semg)
            d2 = pltpu.async_copy(gr_h.at[bi1v.at[pl.ds(goff, 128)]], grb,
                                  semg)
            d3 = pltpu.async_copy(ga_h.at[ancv.at[pl.ds(goff, 128)]], gab,
                                  semg)

            def sub(s, _):
                sl = pl.ds(goff + s * 16, 16)
                a16 = ancv[sl]
                inr = (a16 >= lo) & (a16 < lo + EH)
                dummy = EH + ((goff + s * 16 + lane) & (SG_DUMMY - 1))
                plsc.store_scatter(six, [s * 16 + lane],
                                   jnp.where(inr, a16 - lo, dummy))
                return 0

            lax.fori_loop(0, 8, sub, 0)

            def dsub(f, _):
                r16 = 2 * f + lane // 8
                d16 = plsc.load_gather(dltv, [goff + r16])
                plsc.store_scatter(dwb, [r16, lane % 8], d16 * w8c)
                return 0

            lax.fori_loop(0, 64, dsub, 0)
            d1.wait()
            d2.wait()
            d3.wait()
            pltpu.async_copy(glb, sg.at[six], sems, add=True)
            pltpu.async_copy(grb, sg.at[six], sems, add=True)
            pltpu.async_copy(gab, sg.at[six], sems, add=True)
            pltpu.async_copy(dwb, sg.at[six], sems, add=True)
            return 0

        lax.fori_loop(0, 16, group, 0)
        _drain_scatters()
        return 0

    lax.fori_loop(0, 10, achunk, 0)
    plsc.subcore_barrier()

    loff0 = sid * (EH // NS)
    goff0 = lo + loff0

    def bchunk(kc, _):
        loff = loff0 + kc * 2048
        goff = goff0 + kc * 2048
        pltpu.sync_copy(sg.at[pl.ds(loff, 2048)], outb)
        pltpu.sync_copy(et_h.at[pl.ds(goff, 2048)], etb)

        def sub(f, _):
            r16 = 2 * f + lane // 8
            c16 = lane % 8
            e16 = plsc.load_gather(etb, [r16])
            val = (plsc.load_gather(outb, [r16, c16])
                   + plsc.load_gather(tbv, [e16 * 8 + c16]))
            val = jnp.where(goff + r16 < E, val, 0.0)
            plsc.store_scatter(outb, [r16, c16], val)
            return 0

        lax.fori_loop(0, 1024, sub, 0)
        pltpu.sync_copy(outb, attr_h.at[pl.ds(goff, 2048)])
        return 0

    lax.fori_loop(0, 5, bchunk, 0)


def _k3(anc_p, bi0_p, bi1_p, dlt_p, gl, gr, ga, et_p, tb_f, w8_f):
    scratch = [
        pltpu.VMEM_SHARED((SG_ROWS, 8), F32),
        pltpu.VMEM((128,), F32), pltpu.VMEM((16,), F32),
        pltpu.VMEM((2048,), I32), pltpu.VMEM((2048,), I32),
        pltpu.VMEM((2048,), I32), pltpu.VMEM((2048,), F32),
        pltpu.VMEM((128, 8), F32), pltpu.VMEM((128, 8), F32),
        pltpu.VMEM((128, 8), F32), pltpu.VMEM((128, 8), F32),
        pltpu.VMEM((128,), I32),
        pltpu.VMEM((2048, 8), F32), pltpu.VMEM((2048,), I32),
        pltpu.SemaphoreType.DMA, pltpu.SemaphoreType.DMA,
    ]
    return pl.kernel(_k3_body,
                     out_type=jax.ShapeDtypeStruct((EP, 8), F32),
                     mesh=_mesh(), scratch_types=scratch,
                     compiler_params=_SC_PARAMS)(
        anc_p, bi0_p, bi1_p, dlt_p, gl, gr, ga, et_p, tb_f, w8_f)


def _k4_body(dst_h, as_h, attr_h, p_out,
             psh, dstv, asv, attrv, mb, pidx, pb, *, cp):
    cid = lax.axis_index("c")
    sid = lax.axis_index("s")
    lane = _iota16()
    nlo = cid * NH

    _zero2d(pb, 2048, 8)

    def zchunk(i, _):
        pltpu.sync_copy(pb, psh.at[pl.ds(sid * 8192 + i * 2048, 2048)])
        return 0

    lax.fori_loop(0, 4, zchunk, 0)
    plsc.subcore_barrier()

    ebase = sid * (EP // NS)

    def echunk(kc, _):
        off = ebase + kc * 2048
        pltpu.sync_copy(dst_h.at[pl.ds(off, 2048)], dstv)
        pltpu.sync_copy(as_h.at[pl.ds(off, 2048)], asv)
        pltpu.sync_copy(attr_h.at[pl.ds(off, 2048)], attrv)

        def group(g, _):
            goff = g * 128

            def sub(s, _):
                sl = pl.ds(goff + s * 16, 16)
                d16 = dstv[sl]
                as16 = asv[sl]
                row16 = s * 16 + lane
                inr = (d16 >= nlo) & (d16 < nlo + NH)
                r = (d16 - nlo) * 100 + as16
                q = jnp.where(inr, r // 4,
                              125000 + ((goff + s * 16 + lane) & 4095))
                cb = (r % 4) * 2
                plsc.store_scatter(pidx, [row16], q)
                lrow = goff + s * 16 + lane
                av0 = plsc.load_gather(
                    attrv, [lrow, jnp.full((16,), 2 * cp, I32)])
                av1 = plsc.load_gather(
                    attrv, [lrow, jnp.full((16,), 2 * cp + 1, I32)])
                for c in range(8):
                    cc = jnp.full((16,), c, I32)
                    val = (jnp.where(cb == c, av0, 0.0)
                           + jnp.where(cb + 1 == c, av1, 0.0))
                    plsc.store_scatter(mb, [row16, cc], val)
                return 0

            lax.fori_loop(0, 8, sub, 0)
            pltpu.sync_copy(mb, psh.at[pidx], add=True)
            return 0

        lax.fori_loop(0, 16, group, 0)
        return 0

    lax.fori_loop(0, 10, echunk, 0)
    plsc.subcore_barrier()

    dbase = sid * 8192

    def dchunk(i, _):
        off = dbase + i * 2048
        pltpu.sync_copy(psh.at[pl.ds(off, 2048)], pb)
        pltpu.sync_copy(pb, p_out.at[pl.ds(cid * 131072 + off, 2048)])
        return 0

    lax.fori_loop(0, 4, dchunk, 0)


def _k4(dst_p, asrc, attr8, cp):
    scratch = [
        pltpu.VMEM_SHARED((131072, 8), F32),
        pltpu.VMEM((2048,), I32), pltpu.VMEM((2048,), I32),
        pltpu.VMEM((2048, 8), F32),
        pltpu.VMEM((128, 8), F32), pltpu.VMEM((128,), I32),
        pltpu.VMEM((2048, 8), F32),
    ]
    body = functools.partial(_k4_body, cp=cp)
    return pl.kernel(body,
                     out_type=jax.ShapeDtypeStruct((2 * 131072, 8), F32),
                     mesh=_mesh(), scratch_types=scratch,
                     compiler_params=_SC_PARAMS)(dst_p, asrc, attr8)


def _k5_body(pa, pb, awa, awb, nb0, nW1, u_o):
    out1 = (jnp.dot(pa[...], awa[...], preferred_element_type=F32)
            + jnp.dot(pb[...], awb[...], preferred_element_type=F32)
            + jnp.sum(nb0[...], axis=0, keepdims=True))
    h1 = jnp.tanh(out1)
    vsum = jnp.sum(nW1[...], axis=-1)
    u_o[...] = lax.dot_general(h1, vsum, (((1,), (1,)), ((), ())),
                               preferred_element_type=F32)


def _k5(pa, pb, awa, awb, nb0, nW1):
    grid = (10,)
    return pl.pallas_call(
        _k5_body,
        grid=grid,
        in_specs=[
            pl.BlockSpec((1000, 200), lambda i: (i, 0)),
            pl.BlockSpec((1000, 200), lambda i: (i, 0)),
            pl.BlockSpec((200, 64), lambda i: (0, 0)),
            pl.BlockSpec((200, 64), lambda i: (0, 0)),
            pl.BlockSpec((4, 64), lambda i: (0, 0)),
            pl.BlockSpec((4, 64, 64), lambda i: (0, 0, 0)),
        ],
        out_specs=pl.BlockSpec((1000, 4), lambda i: (i, 0)),
        out_shape=jax.ShapeDtypeStruct((N, 4), F32),
    )(pa, pb, awa, awb, nb0, nW1)


def _k6_body(src_h, attr_h, u_h, part_h, uv, srcv, attrv, accv):
    wid = _wid()
    lane = _iota16()
    pltpu.sync_copy(u_h, uv)
    ebase = wid * (EP // NW)

    def chunk(kc, acc):
        off = ebase + kc * 2048
        pltpu.sync_copy(src_h.at[pl.ds(off, 2048)], srcv)
        pltpu.sync_copy(attr_h.at[pl.ds(off, 2048)], attrv)

        def body(i, acc):
            s16 = srcv[pl.ds(i * 16, 16)]
            row16 = i * 16 + lane
            for ii in range(4):
                ui = plsc.load_gather(uv, [s16, jnp.full((16,), ii, I32)])
                ai = plsc.load_gather(attrv,
                                      [row16, jnp.full((16,), 4 + ii, I32)])
                acc = acc + ui * ai
            return acc

        return lax.fori_loop(0, 128, body, acc)

    acc = lax.fori_loop(0, 5, chunk, jnp.zeros((16,), F32))
    accv[...] = acc
    pltpu.sync_copy(accv, part_h.at[wid])


def _k6(src_p, attr8, u):
    scratch = [
        pltpu.VMEM((N, 4), F32),
        pltpu.VMEM((2048,), I32), pltpu.VMEM((2048, 8), F32),
        pltpu.VMEM((16,), F32),
    ]
    return pl.kernel(_k6_body,
                     out_type=jax.ShapeDtypeStruct((NW, 16), F32),
                     mesh=_mesh(), scratch_types=scratch,
                     compiler_params=_SC_PARAMS)(src_p, attr8, u)


def _k7_body(part, nb1, out):
    out[...] = (jnp.sum(part[...]) / (64.0 * N)
                + jnp.sum(nb1[...]) / 64.0)[None, None]


def _k7(part, nb1):
    return pl.pallas_call(_k7_body,
                          out_shape=jax.ShapeDtypeStruct((1, 1), F32))(part, nb1)


def kernel(atom_type, edge_index, edge_type, bond_anchor, bond_inbound,
           angle_deltas, atom_emb, bond_emb, l_atom_emb, r_atom_emb, orig_emb,
           dest_emb, l_bond_emb, r_bond_emb, anchor_emb, ring_W, ring_b,
           comb_W, comb_b, edge_W0, edge_b0, edge_W1, edge_b1, node_W0,
           node_b0, node_W1, node_b1):
    atom_type = atom_type.astype(I32)
    src = edge_index[0].astype(I32)
    dst = edge_index[1].astype(I32)
    et = edge_type.astype(I32)
    anc = bond_anchor.astype(I32)
    bi0 = bond_inbound[:, 0].astype(I32)
    bi1 = bond_inbound[:, 1].astype(I32)

    epad = EP - E
    src_p = jnp.pad(src, (0, epad))
    dst_p = jnp.pad(dst, (0, epad))
    et_p = jnp.pad(et, (0, epad))
    apad = AP - A
    anc_p = jnp.pad(anc, (0, apad), constant_values=EP)
    bi0_p = jnp.pad(bi0, (0, apad))
    bi1_p = jnp.pad(bi1, (0, apad))
    dlt_p = jnp.pad(angle_deltas, (0, apad))

    tla, tra, tao, td, tb, w8, awa, awb = _k1(
        atom_emb, bond_emb, l_atom_emb, r_atom_emb, orig_emb, dest_emb,
        l_bond_emb, r_bond_emb, anchor_emb, ring_W, ring_b, comb_W, comb_b,
        edge_W0, edge_W1, edge_b0, edge_b1, node_W0)
    asrc, gl, gr, ga = _k2(atom_type, src_p, dst_p, et_p,
                           tla.reshape(-1), tra.reshape(-1), tao.reshape(-1),
                           td.reshape(-1))
    attr8 = _k3(anc_p, bi0_p, bi1_p, dlt_p, gl, gr, ga, et_p,
                tb.reshape(-1), w8.reshape(-1))
    pa4 = _k4(dst_p, asrc, attr8, 0)
    pb4 = _k4(dst_p, asrc, attr8, 1)
    def _unpad(p4):
        return jnp.concatenate(
            [p4[:125000], p4[131072:131072 + 125000]]).reshape(N, 200)
    pa = _unpad(pa4)
    pb = _unpad(pb4)
    u = _k5(pa, pb, awa, awb, node_b0, node_W1)
    part = _k6(src_p, attr8, u)
    out = _k7(part, node_b1)
    return out[0, 0]
